# baseline (speedup 1.0000x reference)
import functools

import jax
import jax.numpy as jnp
from jax import lax
from jax.experimental import pallas as pl
from jax.experimental.pallas import tpu as pltpu
from jax.experimental.pallas import tpu_sc as plsc

TABLE_ROWS = 8192
TABLE_COLS = 8192
N_LOOKUPS = 1048576

NC = 2
NS = 16
NW = NC * NS
L = 16

N_PER_W = N_LOOKUPS // NW
CHUNK = 4096
N_SUPER = N_PER_W // CHUNK
G = 128
K = CHUNK // G

ROWS_PER_W = TABLE_ROWS // NW


def _flatten_body(tab_hbm, flat_hbm, sem):
    wid = lax.axis_index("s") * NC + lax.axis_index("c")
    row0 = wid * ROWS_PER_W

    def fire(r, _):
        row = row0 + r
        pltpu.async_copy(
            tab_hbm.at[row],
            flat_hbm.at[pl.ds(row * TABLE_COLS, TABLE_COLS)],
            sem,
        )
        return 0

    lax.fori_loop(0, ROWS_PER_W, fire, 0)
    pltpu.make_async_copy(
        flat_hbm.at[pl.ds(0, ROWS_PER_W * TABLE_COLS)],
        flat_hbm.at[pl.ds(row0 * TABLE_COLS, ROWS_PER_W * TABLE_COLS)],
        sem,
    ).wait()


def _gather_body(idx_hbm, flat_hbm, out_hbm, idx_stage, fidx, outbuf, sem):
    wid = lax.axis_index("s") * NC + lax.axis_index("c")
    lane = lax.iota(jnp.int32, L)

    def super_chunk(s, _):
        base = wid * N_PER_W + s * CHUNK
        pltpu.sync_copy(idx_hbm.at[pl.ds(base * 2, 2 * CHUNK)], idx_stage)

        def fcomp(j, _):
            ev = lane * 2 + j * (2 * L)
            r = plsc.load_gather(idx_stage, [ev])
            c = plsc.load_gather(idx_stage, [ev + 1])
            fidx[pl.ds(j * L, L)] = (r << 13) + c
            return 0

        lax.fori_loop(0, CHUNK // L, fcomp, 0)

        def fire(k, _):
            pltpu.async_copy(
                flat_hbm.at[fidx.at[pl.ds(k * G, G)]],
                outbuf.at[pl.ds(k * G, G)],
                sem,
            )
            return 0

        lax.fori_loop(0, K, fire, 0)
        pltpu.make_async_copy(flat_hbm.at[pl.ds(0, CHUNK)], outbuf, sem).wait()

        pltpu.sync_copy(outbuf, out_hbm.at[pl.ds(base, CHUNK)])
        return 0

    lax.fori_loop(0, N_SUPER, super_chunk, 0)


@jax.jit
def _run(indices, train_table):
    idx_flat = indices.reshape(2 * N_LOOKUPS)
    mesh = plsc.VectorSubcoreMesh(core_axis_name="c", subcore_axis_name="s")

    flatten = functools.partial(
        pl.kernel,
        mesh=mesh,
        out_type=jax.ShapeDtypeStruct((TABLE_ROWS * TABLE_COLS,), jnp.float32),
        scratch_types=[pltpu.SemaphoreType.DMA],
        compiler_params=pltpu.CompilerParams(
            needs_layout_passes=False, use_tc_tiling_on_sc=False
        ),
    )(_flatten_body)
    tab_lin = flatten(train_table)

    gather = functools.partial(
        pl.kernel,
        mesh=mesh,
        out_type=jax.ShapeDtypeStruct((N_LOOKUPS,), jnp.float32),
        scratch_types=[
            pltpu.VMEM((2 * CHUNK,), jnp.int32),
            pltpu.VMEM((CHUNK,), jnp.int32),
            pltpu.VMEM((CHUNK,), jnp.float32),
            pltpu.SemaphoreType.DMA,
        ],
        compiler_params=pltpu.CompilerParams(needs_layout_passes=False),
    )(_gather_body)
    return gather(idx_flat, tab_lin)


def kernel(indices, train_table):
    return _run(indices.astype(jnp.int32), train_table)


# speedup vs baseline: 5.9248x; 5.9248x over previous
import functools

import jax
import jax.numpy as jnp
from jax import lax
from jax.experimental import pallas as pl
from jax.experimental.pallas import tpu as pltpu
from jax.experimental.pallas import tpu_sc as plsc

TABLE_ROWS = 8192
TABLE_COLS = 8192
N_LOOKUPS = 1048576

NC = 2
NS = 16
NW = NC * NS
L = 16

BLK = 16
N_BLOCKS = TABLE_ROWS * TABLE_COLS // BLK

N_PER_W = N_LOOKUPS // NW
CHUNK = 4096
N_SUPER = N_PER_W // CHUNK
G = 128
K = CHUNK // G


def _gather_body(idx_hbm, tab_hbm, out_hbm, idx_stage, fblk, flo, blocks, outbuf, sem):
    wid = lax.axis_index("s") * NC + lax.axis_index("c")
    lane = lax.iota(jnp.int32, L)

    def super_chunk(s, _):
        base = wid * N_PER_W + s * CHUNK
        pltpu.sync_copy(idx_hbm.at[pl.ds(base * 2, 2 * CHUNK)], idx_stage)

        def fcomp(j, _):
            ev = lane * 2 + j * (2 * L)
            r = plsc.load_gather(idx_stage, [ev])
            c = plsc.load_gather(idx_stage, [ev + 1])
            fblk[pl.ds(j * L, L)] = (r << 9) + (c >> 4)
            flo[pl.ds(j * L, L)] = c & 15
            return 0

        lax.fori_loop(0, CHUNK // L, fcomp, 0)

        def fire(k, _):
            pltpu.async_copy(
                tab_hbm.at[fblk.at[pl.ds(k * G, G)], :],
                blocks.at[pl.ds(k * G, G), :],
                sem,
            )
            return 0

        lax.fori_loop(0, K, fire, 0)
        pltpu.make_async_copy(tab_hbm.at[pl.ds(0, CHUNK), :], blocks, sem).wait()

        def extract(j, _):
            ids = lane + j * L
            lo = flo[pl.ds(j * L, L)]
            outbuf[pl.ds(j * L, L)] = plsc.load_gather(blocks, [ids, lo])
            return 0

        lax.fori_loop(0, CHUNK // L, extract, 0)

        pltpu.sync_copy(outbuf, out_hbm.at[pl.ds(base, CHUNK)])
        return 0

    lax.fori_loop(0, N_SUPER, super_chunk, 0)


@jax.jit
def _run(indices, train_table):
    idx_flat = indices.reshape(2 * N_LOOKUPS)
    tab_blocks = train_table.reshape(N_BLOCKS, BLK)
    mesh = plsc.VectorSubcoreMesh(core_axis_name="c", subcore_axis_name="s")
    gather = functools.partial(
        pl.kernel,
        mesh=mesh,
        out_type=jax.ShapeDtypeStruct((N_LOOKUPS,), jnp.float32),
        scratch_types=[
            pltpu.VMEM((2 * CHUNK,), jnp.int32),
            pltpu.VMEM((CHUNK,), jnp.int32),
            pltpu.VMEM((CHUNK,), jnp.int32),
            pltpu.VMEM((CHUNK, BLK), jnp.float32),
            pltpu.VMEM((CHUNK,), jnp.float32),
            pltpu.SemaphoreType.DMA,
        ],
        compiler_params=pltpu.CompilerParams(
            needs_layout_passes=False, use_tc_tiling_on_sc=False
        ),
    )(_gather_body)
    return gather(idx_flat, tab_blocks)


def kernel(indices, train_table):
    return _run(indices.astype(jnp.int32), train_table)


# speedup vs baseline: 5.9936x; 1.0116x over previous
import functools

import jax
import jax.numpy as jnp
from jax import lax
from jax.experimental import pallas as pl
from jax.experimental.pallas import tpu as pltpu
from jax.experimental.pallas import tpu_sc as plsc

TABLE_ROWS = 8192
TABLE_COLS = 8192
N_LOOKUPS = 1048576

NC = 2
NS = 16
NW = NC * NS
L = 16

N_PER_W = N_LOOKUPS // NW
CHUNK = 4096
N_SUPER = N_PER_W // CHUNK
G = 128
K = CHUNK // G

SPAN = TABLE_ROWS * TABLE_COLS // NW
C2 = 32768
NCH = SPAN // C2
NB = 3


def _flatten_body(tab_hbm, flat_hbm, b0, b1, b2, sem_in, s0, s1, s2):
    wid = lax.axis_index("s") * NC + lax.axis_index("c")
    base = wid * SPAN
    bufs = [b0, b1, b2]
    sems = [s0, s1, s2]

    HW = TABLE_COLS // 2

    def src_at(i):
        row_off = pl.multiple_of((wid * 32 + i // 2) * 8, 8)
        return tab_hbm.at[pl.ds(row_off, 8), pl.ds((i % 2) * HW, HW)]

    def store_chunk(i, b):
        row0 = (wid * 32 + i // 2) * 8
        for rr in range(8):
            pltpu.async_copy(
                bufs[b].at[rr],
                flat_hbm.at[
                    pl.ds((row0 + rr) * TABLE_COLS + (i % 2) * HW, HW)
                ],
                sems[b],
            )

    def drain_chunk(b):
        pltpu.make_async_copy(src_at(0), bufs[b], sems[b]).wait()

    pltpu.async_copy(src_at(0), bufs[0], sem_in)
    for i in range(NCH):
        b = i % NB
        nxt = i + 1
        if nxt < NCH:
            bn = nxt % NB
            if nxt >= NB:
                drain_chunk(bn)
            pltpu.async_copy(src_at(nxt), bufs[bn], sem_in)
        pltpu.make_async_copy(src_at(i), bufs[b], sem_in).wait()
        store_chunk(i, b)
    for i in range(NCH - NB + 1, NCH):
        drain_chunk(i % NB)


def _gather_body(idx_hbm, flat_hbm, out_hbm, idx_stage, fidx, outbuf, sem):
    wid = lax.axis_index("s") * NC + lax.axis_index("c")
    lane = lax.iota(jnp.int32, L)

    def super_chunk(s, _):
        base = wid * N_PER_W + s * CHUNK
        pltpu.sync_copy(idx_hbm.at[pl.ds(base * 2, 2 * CHUNK)], idx_stage)

        def fcomp(j, _):
            ev = lane * 2 + j * (2 * L)
            r = plsc.load_gather(idx_stage, [ev])
            c = plsc.load_gather(idx_stage, [ev + 1])
            fidx[pl.ds(j * L, L)] = (r << 13) + c
            return 0

        lax.fori_loop(0, CHUNK // L, fcomp, 0)

        def fire(k, _):
            pltpu.async_copy(
                flat_hbm.at[fidx.at[pl.ds(k * G, G)]],
                outbuf.at[pl.ds(k * G, G)],
                sem,
            )
            return 0

        lax.fori_loop(0, K, fire, 0)
        pltpu.make_async_copy(flat_hbm.at[pl.ds(0, CHUNK)], outbuf, sem).wait()

        pltpu.sync_copy(outbuf, out_hbm.at[pl.ds(base, CHUNK)])
        return 0

    lax.fori_loop(0, N_SUPER, super_chunk, 0)


@jax.jit
def _run(indices, train_table):
    idx_flat = indices.reshape(2 * N_LOOKUPS)
    mesh = plsc.VectorSubcoreMesh(core_axis_name="c", subcore_axis_name="s")

    flatten = functools.partial(
        pl.kernel,
        mesh=mesh,
        out_type=jax.ShapeDtypeStruct((TABLE_ROWS * TABLE_COLS,), jnp.float32),
        scratch_types=[
            pltpu.VMEM((8, TABLE_COLS // 2), jnp.float32),
            pltpu.VMEM((8, TABLE_COLS // 2), jnp.float32),
            pltpu.VMEM((8, TABLE_COLS // 2), jnp.float32),
            pltpu.SemaphoreType.DMA,
            pltpu.SemaphoreType.DMA,
            pltpu.SemaphoreType.DMA,
            pltpu.SemaphoreType.DMA,
        ],
        compiler_params=pltpu.CompilerParams(needs_layout_passes=False),
    )(_flatten_body)
    tab_lin = flatten(train_table)

    gather = functools.partial(
        pl.kernel,
        mesh=mesh,
        out_type=jax.ShapeDtypeStruct((N_LOOKUPS,), jnp.float32),
        scratch_types=[
            pltpu.VMEM((2 * CHUNK,), jnp.int32),
            pltpu.VMEM((CHUNK,), jnp.int32),
            pltpu.VMEM((CHUNK,), jnp.float32),
            pltpu.SemaphoreType.DMA,
        ],
        compiler_params=pltpu.CompilerParams(needs_layout_passes=False),
    )(_gather_body)
    return gather(idx_flat, tab_lin)


def kernel(indices, train_table):
    return _run(indices.astype(jnp.int32), train_table)


# speedup vs baseline: 6.4539x; 1.0768x over previous
import functools

import jax
import jax.numpy as jnp
from jax import lax
from jax.experimental import pallas as pl
from jax.experimental.pallas import tpu as pltpu
from jax.experimental.pallas import tpu_sc as plsc

TABLE_ROWS = 8192
TABLE_COLS = 8192
N_LOOKUPS = 1048576

NC = 2
NS = 16
NW = NC * NS
L = 16

N_PER_W = N_LOOKUPS // NW
CHUNK = 4096
N_SUPER = N_PER_W // CHUNK
G = 512
K = CHUNK // G


def _body(idx_hbm, tab_hbm, out_hbm, idx_stage, fidx, outbuf, sem):
    wid = lax.axis_index("s") * NC + lax.axis_index("c")
    lane = lax.iota(jnp.int32, L)

    def super_chunk(s, _):
        base = wid * N_PER_W + s * CHUNK
        pltpu.sync_copy(idx_hbm.at[pl.ds(base * 2, 2 * CHUNK)], idx_stage)

        def fcomp(j, _):
            ev = lane * 2 + j * (2 * L)
            r = plsc.load_gather(idx_stage, [ev])
            c = plsc.load_gather(idx_stage, [ev + 1])
            phys = (
                ((r >> 3) << 16)
                + ((c >> 7) << 10)
                + ((r & 7) << 7)
                + (c & 127)
            )
            fidx[pl.ds(j * L, L)] = phys
            return 0

        lax.fori_loop(0, CHUNK // L, fcomp, 0)

        def fire(k, _):
            pltpu.async_copy(
                tab_hbm.at[fidx.at[pl.ds(k * G, G)]],
                outbuf.at[pl.ds(k * G, G)],
                sem,
            )
            return 0

        lax.fori_loop(0, K, fire, 0)
        pltpu.make_async_copy(tab_hbm.at[pl.ds(0, CHUNK)], outbuf, sem).wait()

        pltpu.sync_copy(outbuf, out_hbm.at[pl.ds(base, CHUNK)])
        return 0

    lax.fori_loop(0, N_SUPER, super_chunk, 0)


@jax.jit
def _run(indices, train_table):
    idx_flat = indices.reshape(2 * N_LOOKUPS)
    tab_lin = (
        train_table.reshape(1024, 8, 64, 128)
        .transpose(0, 2, 1, 3)
        .reshape(TABLE_ROWS * TABLE_COLS)
    )
    mesh = plsc.VectorSubcoreMesh(core_axis_name="c", subcore_axis_name="s")
    f = functools.partial(
        pl.kernel,
        mesh=mesh,
        out_type=jax.ShapeDtypeStruct((N_LOOKUPS,), jnp.float32),
        scratch_types=[
            pltpu.VMEM((2 * CHUNK,), jnp.int32),
            pltpu.VMEM((CHUNK,), jnp.int32),
            pltpu.VMEM((CHUNK,), jnp.float32),
            pltpu.SemaphoreType.DMA,
        ],
        compiler_params=pltpu.CompilerParams(needs_layout_passes=False),
    )(_body)
    return f(idx_flat, tab_lin)


def kernel(indices, train_table):
    return _run(indices.astype(jnp.int32), train_table)
